# baseline (speedup 1.0000x reference)
import jax
import jax.numpy as jnp
from jax.experimental import pallas as pl
from jax.experimental.pallas import tpu as pltpu

B = 4096
D = 128
A = 1000
G = 8
K = 16
NG = 1392
C = NG // K


def _body(garr_ref, glo_ref, lens_ref, x_ref, b_ref, *rest):
    w_refs = rest[:K]
    o_ref = rest[K]
    i = pl.program_id(0)
    for k in range(K):
        gid = k * C + i
        lo = glo_ref[gid]
        ln = lens_ref[gid]
        lo_c = jnp.minimum(lo, B - G)
        bidx = garr_ref[gid]
        xg = x_ref[pl.ds(lo_c, G), :]
        z = jax.lax.dot_general(xg, w_refs[k][0], (((1,), (1,)), ((), ())),
                                preferred_element_type=jnp.float32)
        z = z + b_ref[pl.ds(bidx, 1), :]
        pos = jax.lax.broadcasted_iota(jnp.int32, (G, D), 0) + lo_c
        mask = (pos >= lo) & (pos < lo + ln)
        old = o_ref[pl.ds(lo_c, G), :]
        o_ref[pl.ds(lo_c, G), :] = jnp.where(mask, z, old)


def kernel(x, a, weight, bias):
    idx = a[:, 0].astype(jnp.int32)
    iota = jnp.arange(B, dtype=jnp.int32)
    sidx, perm = jax.lax.sort_key_val(idx, iota)
    _, inv_perm = jax.lax.sort_key_val(perm, iota)

    starts = jnp.concatenate([jnp.ones((1,), jnp.bool_),
                              sidx[1:] != sidx[:-1]])
    seg_start = jax.lax.cummax(jnp.where(starts, iota, 0))
    pos_in_run = iota - seg_start
    new_group = starts | (pos_in_run % G == 0)
    g = jnp.cumsum(new_group.astype(jnp.int32)) - 1

    gids = jnp.arange(NG, dtype=jnp.int32)
    glo = jnp.searchsorted(g, gids, side='left').astype(jnp.int32)
    ghi = jnp.searchsorted(g, gids, side='right').astype(jnp.int32)
    lens = ghi - glo
    garr = sidx[jnp.minimum(glo, B - 1)]

    x_s = jnp.take(x, perm, axis=0)

    def w_spec(k):
        return pl.BlockSpec(
            (1, D, D),
            lambda i, g_ref, lo_ref, ln_ref, _k=k: (g_ref[_k * C + i], 0, 0))

    z_s = pl.pallas_call(
        _body,
        grid_spec=pltpu.PrefetchScalarGridSpec(
            num_scalar_prefetch=3,
            grid=(C,),
            in_specs=[
                pl.BlockSpec((B, D), lambda i, g_, l_, n_: (0, 0)),
                pl.BlockSpec((A, D), lambda i, g_, l_, n_: (0, 0)),
            ] + [w_spec(k) for k in range(K)],
            out_specs=pl.BlockSpec((B, D), lambda i, g_, l_, n_: (0, 0)),
        ),
        out_shape=jax.ShapeDtypeStruct((B, D), jnp.float32),
    )(garr, glo, lens, x_s, bias, *([weight] * K))

    return jnp.take(z_s, inv_perm, axis=0)


# speedup vs baseline: 2.7941x; 2.7941x over previous
import jax
import jax.numpy as jnp
from jax.experimental import pallas as pl
from jax.experimental.pallas import tpu as pltpu

B = 4096
D = 128
A = 1000
G = 8
K = 16
NG = 1392
C = NG // K


def _body(garr_ref, glo_ref, lens_ref, x_ref, b_ref, *rest):
    w_refs = rest[:K]
    o_ref = rest[K]
    i = pl.program_id(0)
    for k in range(K):
        gid = k * C + i
        lo = glo_ref[gid]
        ln = lens_ref[gid]
        lo_c = jnp.minimum(lo, B - G)
        bidx = garr_ref[gid]
        xg = x_ref[pl.ds(lo_c, G), :]
        z = jax.lax.dot_general(xg, w_refs[k][0], (((1,), (1,)), ((), ())),
                                preferred_element_type=jnp.float32)
        z = z + b_ref[pl.ds(bidx, 1), :]
        pos = jax.lax.broadcasted_iota(jnp.int32, (G, D), 0) + lo_c
        mask = (pos >= lo) & (pos < lo + ln)
        old = o_ref[pl.ds(lo_c, G), :]
        o_ref[pl.ds(lo_c, G), :] = jnp.where(mask, z, old)


def kernel(x, a, weight, bias):
    idx = a[:, 0].astype(jnp.int32)
    iota = jnp.arange(B, dtype=jnp.int32)
    sidx, perm = jax.lax.sort_key_val(idx, iota)

    starts = jnp.concatenate([jnp.ones((1,), jnp.bool_),
                              sidx[1:] != sidx[:-1]])
    seg_start = jax.lax.cummax(jnp.where(starts, iota, 0))
    pos_in_run = iota - seg_start
    new_group = starts | (pos_in_run % G == 0)
    g = jnp.cumsum(new_group.astype(jnp.int32)) - 1

    key = jnp.where(new_group, g, jnp.int32(1 << 20))
    val = jnp.where(new_group, iota, jnp.int32(B)) + sidx * jnp.int32(8192)
    _, cval = jax.lax.sort_key_val(key, val)
    cval = cval[:NG]
    glo = cval % 8192
    garr = cval // 8192
    lens = jnp.concatenate([glo[1:], jnp.full((1,), B, jnp.int32)]) - glo

    x_s = jnp.take(x, perm, axis=0)

    def w_spec(k):
        return pl.BlockSpec(
            (1, D, D),
            lambda i, g_ref, lo_ref, ln_ref, _k=k: (g_ref[_k * C + i], 0, 0))

    z_s = pl.pallas_call(
        _body,
        grid_spec=pltpu.PrefetchScalarGridSpec(
            num_scalar_prefetch=3,
            grid=(C,),
            in_specs=[
                pl.BlockSpec((B, D), lambda i, g_, l_, n_: (0, 0)),
                pl.BlockSpec((A, D), lambda i, g_, l_, n_: (0, 0)),
            ] + [w_spec(k) for k in range(K)],
            out_specs=pl.BlockSpec((B, D), lambda i, g_, l_, n_: (0, 0)),
        ),
        out_shape=jax.ShapeDtypeStruct((B, D), jnp.float32),
    )(garr, glo, lens, x_s, bias, *([weight] * K))

    return jnp.zeros((B, D), jnp.float32).at[perm].set(z_s)


# speedup vs baseline: 3.2080x; 1.1481x over previous
import functools

import jax
import jax.numpy as jnp
from jax import lax
from jax.experimental import pallas as pl
from jax.experimental.pallas import tpu as pltpu
from jax.experimental.pallas import tpu_sc as plsc

B = 4096
D = 128
A = 1000
G = 8
K = 16
NG = 1392
C = NG // K
P = NG * G

NC = 2
NS = 16
NW = NC * NS
BPW = B // NW


def _route(src_hbm, sidx_hbm, didx_hbm, out_hbm, src_v, dst_v, rows_v, sem):
    wid = lax.axis_index("s") * NC + lax.axis_index("c")
    base = wid * BPW
    pltpu.sync_copy(sidx_hbm.at[pl.ds(base, BPW)], src_v)
    pltpu.sync_copy(didx_hbm.at[pl.ds(base, BPW)], dst_v)
    pltpu.async_copy(src_hbm.at[src_v], rows_v, sem).wait()
    pltpu.async_copy(rows_v, out_hbm.at[dst_v], sem).wait()


def _make_route(n_out):
    mesh = plsc.VectorSubcoreMesh(core_axis_name="c", subcore_axis_name="s")
    return functools.partial(
        pl.kernel, mesh=mesh,
        out_type=jax.ShapeDtypeStruct((n_out, D), jnp.float32),
        scratch_types=[
            pltpu.VMEM((BPW,), jnp.int32),
            pltpu.VMEM((BPW,), jnp.int32),
            pltpu.VMEM((BPW, D), jnp.float32),
            pltpu.SemaphoreType.DMA,
        ],
    )(_route)


def _tc_body(garr_ref, x_ref, b_ref, *rest):
    w_refs = rest[:K]
    o_ref = rest[K]
    i = pl.program_id(0)
    for k in range(K):
        bidx = garr_ref[k * C + i]
        xg = x_ref[k, 0]
        z = jax.lax.dot_general(xg, w_refs[k][0], (((1,), (1,)), ((), ())),
                                preferred_element_type=jnp.float32)
        o_ref[k, 0] = z + b_ref[pl.ds(bidx, 1), :]


def kernel(x, a, weight, bias):
    idx = a[:, 0].astype(jnp.int32)
    iota = jnp.arange(B, dtype=jnp.int32)
    sidx, perm = jax.lax.sort_key_val(idx, iota)

    starts = jnp.concatenate([jnp.ones((1,), jnp.bool_),
                              sidx[1:] != sidx[:-1]])
    seg_start = jax.lax.cummax(jnp.where(starts, iota, 0))
    pos_in_run = iota - seg_start
    new_group = starts | (pos_in_run % G == 0)
    g = jnp.cumsum(new_group.astype(jnp.int32)) - 1
    ppos = g * G + pos_in_run % G

    key = jnp.where(new_group, g, jnp.int32(1 << 20))
    val = jnp.where(new_group, sidx, jnp.int32(0))
    _, garr = jax.lax.sort_key_val(key, val)
    garr = garr[:NG]

    x_pad = _make_route(P)(x, perm, ppos)

    def w_spec(k):
        return pl.BlockSpec(
            (1, D, D),
            lambda i, g_ref, _k=k: (g_ref[_k * C + i], 0, 0))

    z_pad = pl.pallas_call(
        _tc_body,
        grid_spec=pltpu.PrefetchScalarGridSpec(
            num_scalar_prefetch=1,
            grid=(C,),
            in_specs=[
                pl.BlockSpec((K, 1, G, D), lambda i, g_: (0, i, 0, 0)),
                pl.BlockSpec((A, D), lambda i, g_: (0, 0)),
            ] + [w_spec(k) for k in range(K)],
            out_specs=pl.BlockSpec((K, 1, G, D), lambda i, g_: (0, i, 0, 0)),
        ),
        out_shape=jax.ShapeDtypeStruct((K, C, G, D), jnp.float32),
    )(garr, x_pad.reshape(K, C, G, D), bias, *([weight] * K))

    return _make_route(B)(z_pad.reshape(P, D), ppos, perm)
